# Initial kernel scaffold; baseline (speedup 1.0000x reference)
import jax
import jax.numpy as jnp
from jax.experimental import pallas as pl


def kernel(x, edge_index, W1, b1, W2, b2):
    raise NotImplementedError("write your pallas kernel here")


# speedup vs baseline: 17.5991x; 17.5991x over previous
import functools

import jax
import jax.numpy as jnp
from jax import lax
from jax.experimental import pallas as pl
from jax.experimental.pallas import tpu as pltpu
from jax.experimental.pallas import tpu_sc as plsc

N = 10000
E = 320000
D = 128

NC = 2
NS = 16
NW = NC * NS
EPW = E // NW
CH = 80
NCH = EPW // CH
NP = 10240
NPC = NP // CH // NS

BR = 1000


def _mesh():
    return plsc.VectorSubcoreMesh(core_axis_name="c", subcore_axis_name="s")


@functools.partial(
    pl.kernel,
    out_type=jax.ShapeDtypeStruct((NC, NP, D), jnp.float32),
    mesh=_mesh(),
    scratch_types=[
        pltpu.VMEM((NCH, CH), jnp.int32),
        pltpu.VMEM((CH, D), jnp.float32),
        pltpu.VMEM_SHARED((NP, D), jnp.float32),
    ],
)
def _deg_kernel(dst_hbm, ones_hbm, z128_hbm, out_hbm, idx_v, ones_v, acc_sh):
    c = lax.axis_index("c")
    s = lax.axis_index("s")
    w = s * NC + c
    pltpu.sync_copy(dst_hbm.at[w], idx_v)
    pltpu.sync_copy(z128_hbm, ones_v)
    for k in range(NPC):
        q = s * NPC + k
        pltpu.sync_copy(ones_v, acc_sh.at[pl.ds(q * CH, CH)])
    pltpu.sync_copy(ones_hbm, ones_v)
    plsc.subcore_barrier()

    def body(j, carry):
        pltpu.sync_copy(ones_v, acc_sh.at[idx_v.at[j]], add=True)
        return carry

    lax.fori_loop(0, NCH, body, 0)
    plsc.subcore_barrier()
    for k in range(NPC):
        q = s * NPC + k
        pltpu.sync_copy(acc_sh.at[pl.ds(q * CH, CH)], ones_v)
        pltpu.sync_copy(ones_v, out_hbm.at[c, pl.ds(q * CH, CH)])


@functools.partial(
    pl.kernel,
    out_type=jax.ShapeDtypeStruct((NC, NP, D), jnp.float32),
    mesh=_mesh(),
    scratch_types=[
        pltpu.VMEM((NCH, CH), jnp.int32),
        pltpu.VMEM((NCH, CH), jnp.int32),
        pltpu.VMEM((CH, D), jnp.float32),
        pltpu.VMEM_SHARED((NP, D), jnp.float32),
    ],
)
def _agg_kernel(y_hbm, src_hbm, dst_hbm, z128_hbm, out_hbm,
                src_v, dst_v, rows_v, acc_sh):
    c = lax.axis_index("c")
    s = lax.axis_index("s")
    w = s * NC + c
    pltpu.sync_copy(src_hbm.at[w], src_v)
    pltpu.sync_copy(dst_hbm.at[w], dst_v)
    pltpu.sync_copy(z128_hbm, rows_v)
    for k in range(NPC):
        q = s * NPC + k
        pltpu.sync_copy(rows_v, acc_sh.at[pl.ds(q * CH, CH)])

    plsc.subcore_barrier()

    def body(j, carry):
        pltpu.sync_copy(y_hbm.at[src_v.at[j]], rows_v)
        pltpu.sync_copy(rows_v, acc_sh.at[dst_v.at[j]], add=True)
        return carry

    lax.fori_loop(0, NCH, body, 0)
    plsc.subcore_barrier()
    for k in range(NPC):
        q = s * NPC + k
        pltpu.sync_copy(acc_sh.at[pl.ds(q * CH, CH)], rows_v)
        pltpu.sync_copy(rows_v, out_hbm.at[c, pl.ds(q * CH, CH)])


def _dinv_from(degp_ref):
    deg = degp_ref[0] + degp_ref[1] + 1.0
    return jnp.where(deg > 0, lax.rsqrt(deg), 0.0)


def _tc1_body(degp_ref, x_ref, w_ref, o_ref):
    dinv = _dinv_from(degp_ref)
    o_ref[...] = dinv * jnp.dot(x_ref[...], w_ref[...],
                                preferred_element_type=jnp.float32)


_tc1 = pl.pallas_call(
    _tc1_body,
    grid=(N // BR,),
    in_specs=[
        pl.BlockSpec((2, BR, 1), lambda i: (0, i, 0)),
        pl.BlockSpec((BR, D), lambda i: (i, 0)),
        pl.BlockSpec((D, D), lambda i: (0, 0)),
    ],
    out_specs=pl.BlockSpec((BR, D), lambda i: (i, 0)),
    out_shape=jax.ShapeDtypeStruct((N, D), jnp.float32),
)


def _tc2_body(degp_ref, s_ref, y1_ref, b1_ref, w2_ref, o_ref):
    dinv = _dinv_from(degp_ref)
    sv = s_ref[...]
    h = jnp.maximum(dinv * (sv[0] + sv[1] + y1_ref[...]) + b1_ref[...], 0.0)
    o_ref[...] = dinv * jnp.dot(h, w2_ref[...],
                                preferred_element_type=jnp.float32)


_tc2 = pl.pallas_call(
    _tc2_body,
    grid=(N // BR,),
    in_specs=[
        pl.BlockSpec((2, BR, 1), lambda i: (0, i, 0)),
        pl.BlockSpec((2, BR, D), lambda i: (0, i, 0)),
        pl.BlockSpec((BR, D), lambda i: (i, 0)),
        pl.BlockSpec((1, D), lambda i: (0, 0)),
        pl.BlockSpec((D, D), lambda i: (0, 0)),
    ],
    out_specs=pl.BlockSpec((BR, D), lambda i: (i, 0)),
    out_shape=jax.ShapeDtypeStruct((N, D), jnp.float32),
)


def _tc3_body(degp_ref, s_ref, y2_ref, b2_ref, o_ref):
    dinv = _dinv_from(degp_ref)
    sv = s_ref[...]
    o_ref[...] = dinv * (sv[0] + sv[1] + y2_ref[...]) + b2_ref[...]


_tc3 = pl.pallas_call(
    _tc3_body,
    grid=(N // BR,),
    in_specs=[
        pl.BlockSpec((2, BR, 1), lambda i: (0, i, 0)),
        pl.BlockSpec((2, BR, D), lambda i: (0, i, 0)),
        pl.BlockSpec((BR, D), lambda i: (i, 0)),
        pl.BlockSpec((1, D), lambda i: (0, 0)),
    ],
    out_specs=pl.BlockSpec((BR, D), lambda i: (i, 0)),
    out_shape=jax.ShapeDtypeStruct((N, D), jnp.float32),
)


def kernel(x, edge_index, W1, b1, W2, b2):
    ei = edge_index.astype(jnp.int32)
    src3 = ei[0].reshape(NW, NCH, CH)
    dst3 = ei[1].reshape(NW, NCH, CH)
    ones128 = jnp.ones((CH, D), jnp.float32)
    z128 = jnp.zeros((CH, D), jnp.float32)

    degp = _deg_kernel(dst3, ones128, z128)
    degc = degp[:, :N, 0:1]
    y1 = _tc1(degc, x, W1)
    s1 = _agg_kernel(y1, src3, dst3, z128)
    y2 = _tc2(degc, s1[:, :N], y1, b1.reshape(1, D), W2)
    s2 = _agg_kernel(y2, src3, dst3, z128)
    out = _tc3(degc, s2[:, :N], y2, b2.reshape(1, D))
    return out


# speedup vs baseline: 21.4412x; 1.2183x over previous
import functools

import jax
import jax.numpy as jnp
from jax import lax
from jax.experimental import pallas as pl
from jax.experimental.pallas import tpu as pltpu
from jax.experimental.pallas import tpu_sc as plsc

N = 10000
E = 320000
D = 128

NC = 2
NS = 16
NW = NC * NS
EPW = E // NW
CH = 80
PH = 64
NCH = EPW // CH
NP = 10240
NPC = NP // CH // NS

BR = 1000


def _mesh():
    return plsc.VectorSubcoreMesh(core_axis_name="c", subcore_axis_name="s")


@functools.partial(
    pl.kernel,
    out_type=jax.ShapeDtypeStruct((NC, NP, D), jnp.float32),
    mesh=_mesh(),
    scratch_types=[
        pltpu.VMEM((NCH, CH), jnp.int32),
        pltpu.VMEM((CH, D), jnp.float32),
        pltpu.VMEM_SHARED((NP, D), jnp.float32),
        pltpu.SemaphoreType.DMA,
    ],
)
def _deg_kernel(dst_hbm, ones_hbm, z128_hbm, out_hbm, idx_v, ones_v, acc_sh,
                ssem):
    c = lax.axis_index("c")
    s = lax.axis_index("s")
    w = s * NC + c
    pltpu.sync_copy(dst_hbm.at[w], idx_v)
    pltpu.sync_copy(z128_hbm, ones_v)
    for k in range(NPC):
        q = s * NPC + k
        pltpu.sync_copy(ones_v, acc_sh.at[pl.ds(q * CH, CH)])
    pltpu.sync_copy(ones_hbm, ones_v)
    plsc.subcore_barrier()

    pltpu.async_copy(ones_v, acc_sh.at[idx_v.at[0]], ssem, add=True)
    pltpu.async_copy(ones_v, acc_sh.at[idx_v.at[1]], ssem, add=True)

    def body(j, carry):
        pltpu.make_async_copy(ones_v, acc_sh.at[idx_v.at[j]], ssem).wait()
        pltpu.async_copy(ones_v, acc_sh.at[idx_v.at[j + 2]], ssem, add=True)
        return carry

    lax.fori_loop(0, NCH - 2, body, 0)
    for j in (NCH - 2, NCH - 1):
        pltpu.make_async_copy(ones_v, acc_sh.at[idx_v.at[j]], ssem).wait()
    plsc.subcore_barrier()
    for k in range(NPC):
        q = s * NPC + k
        pltpu.sync_copy(acc_sh.at[pl.ds(q * CH, CH)], ones_v)
        pltpu.sync_copy(ones_v, out_hbm.at[c, pl.ds(q * CH, CH)])


@functools.partial(
    pl.kernel,
    out_type=jax.ShapeDtypeStruct((NC, NP, D), jnp.float32),
    mesh=_mesh(),
    scratch_types=[
        pltpu.VMEM((PH, CH), jnp.int32),
        pltpu.VMEM((PH, CH), jnp.int32),
        pltpu.VMEM((CH, D), jnp.float32),
        pltpu.VMEM((CH, D), jnp.float32),
        pltpu.VMEM_SHARED((NP, D), jnp.float32),
        pltpu.SemaphoreType.DMA,
        pltpu.SemaphoreType.DMA,
        pltpu.SemaphoreType.DMA,
        pltpu.SemaphoreType.DMA,
    ],
)
def _agg_kernel(y_hbm, src_hbm, dst_hbm, z128_hbm, out_hbm,
                src_v, dst_v, rows_v, rows2_v, acc_sh, gsa, gsb, ssa, ssb):
    c = lax.axis_index("c")
    s = lax.axis_index("s")
    w = s * NC + c

    def load_idx(ph0, cnt):
        pltpu.sync_copy(src_hbm.at[w, pl.ds(ph0, cnt)], src_v.at[pl.ds(0, cnt)])
        pltpu.sync_copy(dst_hbm.at[w, pl.ds(ph0, cnt)], dst_v.at[pl.ds(0, cnt)])

    load_idx(0, PH)
    pltpu.sync_copy(z128_hbm, rows_v)
    for k in range(NPC):
        q = s * NPC + k
        pltpu.sync_copy(rows_v, acc_sh.at[pl.ds(q * CH, CH)])

    plsc.subcore_barrier()

    def pipe(n):
        pltpu.async_copy(y_hbm.at[src_v.at[0]], rows_v, gsa)
        pltpu.async_copy(y_hbm.at[src_v.at[1]], rows2_v, gsb)

        def body(t, carry):
            a = 2 * t
            b = a + 1
            pltpu.make_async_copy(y_hbm.at[src_v.at[a]], rows_v, gsa).wait()
            pltpu.async_copy(rows_v, acc_sh.at[dst_v.at[a]], ssa, add=True)
            pltpu.make_async_copy(y_hbm.at[src_v.at[b]], rows2_v, gsb).wait()
            pltpu.async_copy(rows2_v, acc_sh.at[dst_v.at[b]], ssb, add=True)
            pltpu.make_async_copy(rows_v, acc_sh.at[dst_v.at[a]], ssa).wait()
            pltpu.async_copy(y_hbm.at[src_v.at[a + 2]], rows_v, gsa)
            pltpu.make_async_copy(rows2_v, acc_sh.at[dst_v.at[b]], ssb).wait()
            pltpu.async_copy(y_hbm.at[src_v.at[b + 2]], rows2_v, gsb)
            return carry

        if n % 2 == 0:
            lax.fori_loop(0, (n - 2) // 2, body, 0)
            a, b = n - 2, n - 1
            pltpu.make_async_copy(y_hbm.at[src_v.at[a]], rows_v, gsa).wait()
            pltpu.async_copy(rows_v, acc_sh.at[dst_v.at[a]], ssa, add=True)
            pltpu.make_async_copy(y_hbm.at[src_v.at[b]], rows2_v, gsb).wait()
            pltpu.async_copy(rows2_v, acc_sh.at[dst_v.at[b]], ssb, add=True)
            pltpu.make_async_copy(rows_v, acc_sh.at[dst_v.at[a]], ssa).wait()
            pltpu.make_async_copy(rows2_v, acc_sh.at[dst_v.at[b]], ssb).wait()
        else:
            lax.fori_loop(0, (n - 3) // 2, body, 0)
            a, b, z = n - 3, n - 2, n - 1
            pltpu.make_async_copy(y_hbm.at[src_v.at[a]], rows_v, gsa).wait()
            pltpu.async_copy(rows_v, acc_sh.at[dst_v.at[a]], ssa, add=True)
            pltpu.make_async_copy(y_hbm.at[src_v.at[b]], rows2_v, gsb).wait()
            pltpu.async_copy(rows2_v, acc_sh.at[dst_v.at[b]], ssb, add=True)
            pltpu.make_async_copy(rows_v, acc_sh.at[dst_v.at[a]], ssa).wait()
            pltpu.async_copy(y_hbm.at[src_v.at[z]], rows_v, gsa)
            pltpu.make_async_copy(y_hbm.at[src_v.at[z]], rows_v, gsa).wait()
            pltpu.async_copy(rows_v, acc_sh.at[dst_v.at[z]], ssa, add=True)
            pltpu.make_async_copy(rows_v, acc_sh.at[dst_v.at[z]], ssa).wait()
            pltpu.make_async_copy(rows2_v, acc_sh.at[dst_v.at[b]], ssb).wait()

    pipe(PH)
    load_idx(PH, NCH - PH)
    pipe(NCH - PH)
    plsc.subcore_barrier()
    for k in range(NPC):
        q = s * NPC + k
        pltpu.sync_copy(acc_sh.at[pl.ds(q * CH, CH)], rows_v)
        pltpu.sync_copy(rows_v, out_hbm.at[c, pl.ds(q * CH, CH)])


def _dinv_from(degp_ref):
    deg = degp_ref[0] + degp_ref[1] + 1.0
    return jnp.where(deg > 0, lax.rsqrt(deg), 0.0)


def _tc1_body(degp_ref, x_ref, w_ref, o_ref):
    dinv = _dinv_from(degp_ref)
    o_ref[...] = dinv * jnp.dot(x_ref[...], w_ref[...],
                                preferred_element_type=jnp.float32)


_tc1 = pl.pallas_call(
    _tc1_body,
    grid=(N // BR,),
    in_specs=[
        pl.BlockSpec((2, BR, 1), lambda i: (0, i, 0)),
        pl.BlockSpec((BR, D), lambda i: (i, 0)),
        pl.BlockSpec((D, D), lambda i: (0, 0)),
    ],
    out_specs=pl.BlockSpec((BR, D), lambda i: (i, 0)),
    out_shape=jax.ShapeDtypeStruct((N, D), jnp.float32),
)


def _tc2_body(degp_ref, s_ref, y1_ref, b1_ref, w2_ref, o_ref):
    dinv = _dinv_from(degp_ref)
    sv = s_ref[...]
    h = jnp.maximum(dinv * (sv[0] + sv[1] + y1_ref[...]) + b1_ref[...], 0.0)
    o_ref[...] = dinv * jnp.dot(h, w2_ref[...],
                                preferred_element_type=jnp.float32)


_tc2 = pl.pallas_call(
    _tc2_body,
    grid=(N // BR,),
    in_specs=[
        pl.BlockSpec((2, BR, 1), lambda i: (0, i, 0)),
        pl.BlockSpec((2, BR, D), lambda i: (0, i, 0)),
        pl.BlockSpec((BR, D), lambda i: (i, 0)),
        pl.BlockSpec((1, D), lambda i: (0, 0)),
        pl.BlockSpec((D, D), lambda i: (0, 0)),
    ],
    out_specs=pl.BlockSpec((BR, D), lambda i: (i, 0)),
    out_shape=jax.ShapeDtypeStruct((N, D), jnp.float32),
)


def _tc3_body(degp_ref, s_ref, y2_ref, b2_ref, o_ref):
    dinv = _dinv_from(degp_ref)
    sv = s_ref[...]
    o_ref[...] = dinv * (sv[0] + sv[1] + y2_ref[...]) + b2_ref[...]


_tc3 = pl.pallas_call(
    _tc3_body,
    grid=(N // BR,),
    in_specs=[
        pl.BlockSpec((2, BR, 1), lambda i: (0, i, 0)),
        pl.BlockSpec((2, BR, D), lambda i: (0, i, 0)),
        pl.BlockSpec((BR, D), lambda i: (i, 0)),
        pl.BlockSpec((1, D), lambda i: (0, 0)),
    ],
    out_specs=pl.BlockSpec((BR, D), lambda i: (i, 0)),
    out_shape=jax.ShapeDtypeStruct((N, D), jnp.float32),
)


def kernel(x, edge_index, W1, b1, W2, b2):
    ei = edge_index.astype(jnp.int32)
    src3 = ei[0].reshape(NW, NCH, CH)
    dst3 = ei[1].reshape(NW, NCH, CH)
    ones128 = jnp.ones((CH, D), jnp.float32)
    z128 = jnp.zeros((CH, D), jnp.float32)

    degp = _deg_kernel(dst3, ones128, z128)
    degc = degp[:, :N, 0:1]
    y1 = _tc1(degc, x, W1)
    s1 = _agg_kernel(y1, src3, dst3, z128)
    y2 = _tc2(degc, s1[:, :N], y1, b1.reshape(1, D), W2)
    s2 = _agg_kernel(y2, src3, dst3, z128)
    out = _tc3(degc, s2[:, :N], y2, b2.reshape(1, D))
    return out


# speedup vs baseline: 24.2457x; 1.1308x over previous
import functools

import jax
import jax.numpy as jnp
from jax import lax
from jax.experimental import pallas as pl
from jax.experimental.pallas import tpu as pltpu
from jax.experimental.pallas import tpu_sc as plsc

N = 10000
E = 320000
D = 128

NC = 2
NS = 16
NW = NC * NS
EPW = E // NW
CH = 80
PH = 48
NCH = EPW // CH
NP = 10240
NPC = NP // CH // NS

BR = 1000


def _mesh():
    return plsc.VectorSubcoreMesh(core_axis_name="c", subcore_axis_name="s")


@functools.partial(
    pl.kernel,
    out_type=jax.ShapeDtypeStruct((NC, NP, D), jnp.float32),
    mesh=_mesh(),
    scratch_types=[
        pltpu.VMEM((NCH, CH), jnp.int32),
        pltpu.VMEM((CH, D), jnp.float32),
        pltpu.VMEM_SHARED((NP, D), jnp.float32),
        pltpu.SemaphoreType.DMA,
    ],
)
def _deg_kernel(dst_hbm, ones_hbm, z128_hbm, out_hbm, idx_v, ones_v, acc_sh,
                ssem):
    c = lax.axis_index("c")
    s = lax.axis_index("s")
    w = s * NC + c
    pltpu.sync_copy(dst_hbm.at[w], idx_v)
    pltpu.sync_copy(z128_hbm, ones_v)
    for k in range(NPC):
        q = s * NPC + k
        pltpu.sync_copy(ones_v, acc_sh.at[pl.ds(q * CH, CH)])
    pltpu.sync_copy(ones_hbm, ones_v)
    plsc.subcore_barrier()

    pltpu.async_copy(ones_v, acc_sh.at[idx_v.at[0]], ssem, add=True)
    pltpu.async_copy(ones_v, acc_sh.at[idx_v.at[1]], ssem, add=True)

    def body(j, carry):
        pltpu.make_async_copy(ones_v, acc_sh.at[idx_v.at[j]], ssem).wait()
        pltpu.async_copy(ones_v, acc_sh.at[idx_v.at[j + 2]], ssem, add=True)
        return carry

    lax.fori_loop(0, NCH - 2, body, 0)
    for j in (NCH - 2, NCH - 1):
        pltpu.make_async_copy(ones_v, acc_sh.at[idx_v.at[j]], ssem).wait()
    plsc.subcore_barrier()
    for k in range(NPC):
        q = s * NPC + k
        pltpu.sync_copy(acc_sh.at[pl.ds(q * CH, CH)], ones_v)
        pltpu.sync_copy(ones_v, out_hbm.at[c, pl.ds(q * CH, CH)])


@functools.partial(
    pl.kernel,
    out_type=jax.ShapeDtypeStruct((NC, NP, D), jnp.float32),
    mesh=_mesh(),
    scratch_types=[
        pltpu.VMEM((PH, CH), jnp.int32),
        pltpu.VMEM((PH, CH), jnp.int32),
        pltpu.VMEM((CH, D), jnp.float32),
        pltpu.VMEM((CH, D), jnp.float32),
        pltpu.VMEM((CH, D), jnp.float32),
        pltpu.VMEM_SHARED((NP, D), jnp.float32),
        pltpu.SemaphoreType.DMA,
        pltpu.SemaphoreType.DMA,
        pltpu.SemaphoreType.DMA,
        pltpu.SemaphoreType.DMA,
        pltpu.SemaphoreType.DMA,
        pltpu.SemaphoreType.DMA,
    ],
)
def _agg_kernel(y_hbm, src_hbm, dst_hbm, z128_hbm, out_hbm,
                src_v, dst_v, rows_v, rows2_v, rows3_v, acc_sh,
                gsa, gsb, gsc, ssa, ssb, ssc):
    c = lax.axis_index("c")
    s = lax.axis_index("s")
    w = s * NC + c

    def load_idx(ph0, cnt):
        pltpu.sync_copy(src_hbm.at[w, pl.ds(ph0, cnt)], src_v.at[pl.ds(0, cnt)])
        pltpu.sync_copy(dst_hbm.at[w, pl.ds(ph0, cnt)], dst_v.at[pl.ds(0, cnt)])

    load_idx(0, PH)
    pltpu.sync_copy(z128_hbm, rows_v)
    for k in range(NPC):
        q = s * NPC + k
        pltpu.sync_copy(rows_v, acc_sh.at[pl.ds(q * CH, CH)])

    plsc.subcore_barrier()

    def pipe(n):
        B = (rows_v, gsa, ssa), (rows2_v, gsb, ssb), (rows3_v, gsc, ssc)
        for r in range(3):
            pltpu.async_copy(y_hbm.at[src_v.at[r]], B[r][0], B[r][1])

        def body(t, carry):
            base = 3 * t
            for r in range(3):
                buf, gs, ss = B[r]
                j = base + r
                pltpu.make_async_copy(y_hbm.at[src_v.at[j]], buf, gs).wait()
                pltpu.async_copy(buf, acc_sh.at[dst_v.at[j]], ss, add=True)
            for r in range(3):
                buf, gs, ss = B[r]
                j = base + r
                pltpu.make_async_copy(buf, acc_sh.at[dst_v.at[j]], ss).wait()
                pltpu.async_copy(y_hbm.at[src_v.at[j + 3]], buf, gs)
            return carry

        T = (n - 3) // 3
        lax.fori_loop(0, T, body, 0)
        rem = list(range(3 * T, n))
        for r in rem:
            buf, gs, ss = B[r % 3]
            if r >= 3 * T + 3:
                pltpu.make_async_copy(buf, acc_sh.at[dst_v.at[r - 3]], ss).wait()
                pltpu.async_copy(y_hbm.at[src_v.at[r]], buf, gs)
            pltpu.make_async_copy(y_hbm.at[src_v.at[r]], buf, gs).wait()
            pltpu.async_copy(buf, acc_sh.at[dst_v.at[r]], ss, add=True)
        last = {}
        for r in rem:
            last[r % 3] = r
        for k in sorted(last):
            buf, gs, ss = B[k]
            pltpu.make_async_copy(buf, acc_sh.at[dst_v.at[last[k]]], ss).wait()

    pipe(PH)
    load_idx(PH, PH)
    pipe(PH)
    load_idx(2 * PH, NCH - 2 * PH)
    pipe(NCH - 2 * PH)
    plsc.subcore_barrier()
    for k in range(NPC):
        q = s * NPC + k
        pltpu.sync_copy(acc_sh.at[pl.ds(q * CH, CH)], rows_v)
        pltpu.sync_copy(rows_v, out_hbm.at[c, pl.ds(q * CH, CH)])


def _dinv_from(degp_ref):
    deg = degp_ref[0] + degp_ref[1] + 1.0
    return jnp.where(deg > 0, lax.rsqrt(deg), 0.0)


def _tc1_body(degp_ref, x_ref, w_ref, o_ref):
    dinv = _dinv_from(degp_ref)
    o_ref[...] = dinv * jnp.dot(x_ref[...], w_ref[...],
                                preferred_element_type=jnp.float32)


_tc1 = pl.pallas_call(
    _tc1_body,
    grid=(N // BR,),
    in_specs=[
        pl.BlockSpec((2, BR, 1), lambda i: (0, i, 0)),
        pl.BlockSpec((BR, D), lambda i: (i, 0)),
        pl.BlockSpec((D, D), lambda i: (0, 0)),
    ],
    out_specs=pl.BlockSpec((BR, D), lambda i: (i, 0)),
    out_shape=jax.ShapeDtypeStruct((N, D), jnp.float32),
)


def _tc2_body(degp_ref, s_ref, y1_ref, b1_ref, w2_ref, o_ref):
    dinv = _dinv_from(degp_ref)
    sv = s_ref[...]
    h = jnp.maximum(dinv * (sv[0] + sv[1] + y1_ref[...]) + b1_ref[...], 0.0)
    o_ref[...] = dinv * jnp.dot(h, w2_ref[...],
                                preferred_element_type=jnp.float32)


_tc2 = pl.pallas_call(
    _tc2_body,
    grid=(N // BR,),
    in_specs=[
        pl.BlockSpec((2, BR, 1), lambda i: (0, i, 0)),
        pl.BlockSpec((2, BR, D), lambda i: (0, i, 0)),
        pl.BlockSpec((BR, D), lambda i: (i, 0)),
        pl.BlockSpec((1, D), lambda i: (0, 0)),
        pl.BlockSpec((D, D), lambda i: (0, 0)),
    ],
    out_specs=pl.BlockSpec((BR, D), lambda i: (i, 0)),
    out_shape=jax.ShapeDtypeStruct((N, D), jnp.float32),
)


def _tc3_body(degp_ref, s_ref, y2_ref, b2_ref, o_ref):
    dinv = _dinv_from(degp_ref)
    sv = s_ref[...]
    o_ref[...] = dinv * (sv[0] + sv[1] + y2_ref[...]) + b2_ref[...]


_tc3 = pl.pallas_call(
    _tc3_body,
    grid=(N // BR,),
    in_specs=[
        pl.BlockSpec((2, BR, 1), lambda i: (0, i, 0)),
        pl.BlockSpec((2, BR, D), lambda i: (0, i, 0)),
        pl.BlockSpec((BR, D), lambda i: (i, 0)),
        pl.BlockSpec((1, D), lambda i: (0, 0)),
    ],
    out_specs=pl.BlockSpec((BR, D), lambda i: (i, 0)),
    out_shape=jax.ShapeDtypeStruct((N, D), jnp.float32),
)


def kernel(x, edge_index, W1, b1, W2, b2):
    ei = edge_index.astype(jnp.int32)
    src3 = ei[0].reshape(NW, NCH, CH)
    dst3 = ei[1].reshape(NW, NCH, CH)
    ones128 = jnp.ones((CH, D), jnp.float32)
    z128 = jnp.zeros((CH, D), jnp.float32)

    degp = _deg_kernel(dst3, ones128, z128)
    degc = degp[:, :N, 0:1]
    y1 = _tc1(degc, x, W1)
    s1 = _agg_kernel(y1, src3, dst3, z128)
    y2 = _tc2(degc, s1[:, :N], y1, b1.reshape(1, D), W2)
    s2 = _agg_kernel(y2, src3, dst3, z128)
    out = _tc3(degc, s2[:, :N], y2, b2.reshape(1, D))
    return out


# speedup vs baseline: 24.7219x; 1.0196x over previous
import functools

import jax
import jax.numpy as jnp
from jax import lax
from jax.experimental import pallas as pl
from jax.experimental.pallas import tpu as pltpu
from jax.experimental.pallas import tpu_sc as plsc

N = 10000
E = 320000
D = 128

NC = 2
NS = 16
NW = NC * NS
EPW = E // NW
CH = 80
PH = 48
NCH = EPW // CH
NP = 10240
NPC = NP // CH // NS

BR = 1000


def _mesh():
    return plsc.VectorSubcoreMesh(core_axis_name="c", subcore_axis_name="s")


@functools.partial(
    pl.kernel,
    out_type=jax.ShapeDtypeStruct((NC, NP, D), jnp.float32),
    mesh=_mesh(),
    scratch_types=[
        pltpu.VMEM((NCH, CH), jnp.int32),
        pltpu.VMEM((CH, D), jnp.float32),
        pltpu.VMEM((CH, D), jnp.float32),
        pltpu.VMEM_SHARED((NP, D), jnp.float32),
        pltpu.SemaphoreType.DMA,
    ],
)
def _deg_kernel(dst_hbm, ones_hbm, z128_hbm, out_hbm, idx_v, ones_v, ones2_v,
                acc_sh, ssem):
    c = lax.axis_index("c")
    s = lax.axis_index("s")
    w = s * NC + c
    pltpu.sync_copy(dst_hbm.at[w], idx_v)
    pltpu.sync_copy(z128_hbm, ones_v)
    for k in range(NPC):
        q = s * NPC + k
        pltpu.async_copy(ones_v, acc_sh.at[pl.ds(q * CH, CH)], ssem)
    for k in range(NPC):
        q = s * NPC + k
        pltpu.make_async_copy(ones_v, acc_sh.at[pl.ds(q * CH, CH)], ssem).wait()
    pltpu.sync_copy(ones_hbm, ones_v)
    plsc.subcore_barrier()

    pltpu.async_copy(ones_v, acc_sh.at[idx_v.at[0]], ssem, add=True)
    pltpu.async_copy(ones_v, acc_sh.at[idx_v.at[1]], ssem, add=True)

    def body(j, carry):
        pltpu.make_async_copy(ones_v, acc_sh.at[idx_v.at[j]], ssem).wait()
        pltpu.async_copy(ones_v, acc_sh.at[idx_v.at[j + 2]], ssem, add=True)
        return carry

    lax.fori_loop(0, NCH - 2, body, 0)
    for j in (NCH - 2, NCH - 1):
        pltpu.make_async_copy(ones_v, acc_sh.at[idx_v.at[j]], ssem).wait()
    plsc.subcore_barrier()
    for k in range(NPC):
        q = s * NPC + k
        buf = ones_v if k % 2 == 0 else ones2_v
        if k >= 2:
            p = s * NPC + k - 2
            pltpu.make_async_copy(buf, out_hbm.at[c, pl.ds(p * CH, CH)], ssem).wait()
        pltpu.sync_copy(acc_sh.at[pl.ds(q * CH, CH)], buf)
        pltpu.async_copy(buf, out_hbm.at[c, pl.ds(q * CH, CH)], ssem)
    for k in (NPC - 2, NPC - 1):
        q = s * NPC + k
        buf = ones_v if k % 2 == 0 else ones2_v
        pltpu.make_async_copy(buf, out_hbm.at[c, pl.ds(q * CH, CH)], ssem).wait()


@functools.partial(
    pl.kernel,
    out_type=jax.ShapeDtypeStruct((NC, NP, D), jnp.float32),
    mesh=_mesh(),
    scratch_types=[
        pltpu.VMEM((PH, CH), jnp.int32),
        pltpu.VMEM((PH, CH), jnp.int32),
        pltpu.VMEM((CH, D), jnp.float32),
        pltpu.VMEM((CH, D), jnp.float32),
        pltpu.VMEM((CH, D), jnp.float32),
        pltpu.VMEM_SHARED((NP, D), jnp.float32),
        pltpu.SemaphoreType.DMA,
        pltpu.SemaphoreType.DMA,
        pltpu.SemaphoreType.DMA,
        pltpu.SemaphoreType.DMA,
        pltpu.SemaphoreType.DMA,
        pltpu.SemaphoreType.DMA,
    ],
)
def _agg_kernel(y_hbm, src_hbm, dst_hbm, z128_hbm, out_hbm,
                src_v, dst_v, rows_v, rows2_v, rows3_v, acc_sh,
                gsa, gsb, gsc, ssa, ssb, ssc):
    c = lax.axis_index("c")
    s = lax.axis_index("s")
    w = s * NC + c

    def load_idx(ph0, cnt):
        pltpu.sync_copy(src_hbm.at[w, pl.ds(ph0, cnt)], src_v.at[pl.ds(0, cnt)])
        pltpu.sync_copy(dst_hbm.at[w, pl.ds(ph0, cnt)], dst_v.at[pl.ds(0, cnt)])

    load_idx(0, PH)
    pltpu.sync_copy(z128_hbm, rows_v)
    for k in range(NPC):
        q = s * NPC + k
        pltpu.async_copy(rows_v, acc_sh.at[pl.ds(q * CH, CH)], ssa)
    for k in range(NPC):
        q = s * NPC + k
        pltpu.make_async_copy(rows_v, acc_sh.at[pl.ds(q * CH, CH)], ssa).wait()

    plsc.subcore_barrier()

    def pipe(n):
        B = (rows_v, gsa, ssa), (rows2_v, gsb, ssb), (rows3_v, gsc, ssc)
        for r in range(3):
            pltpu.async_copy(y_hbm.at[src_v.at[r]], B[r][0], B[r][1])

        def body(t, carry):
            base = 3 * t
            for r in range(3):
                buf, gs, ss = B[r]
                j = base + r
                pltpu.make_async_copy(y_hbm.at[src_v.at[j]], buf, gs).wait()
                pltpu.async_copy(buf, acc_sh.at[dst_v.at[j]], ss, add=True)
            for r in range(3):
                buf, gs, ss = B[r]
                j = base + r
                pltpu.make_async_copy(buf, acc_sh.at[dst_v.at[j]], ss).wait()
                pltpu.async_copy(y_hbm.at[src_v.at[j + 3]], buf, gs)
            return carry

        T = (n - 3) // 3
        lax.fori_loop(0, T, body, 0)
        rem = list(range(3 * T, n))
        for r in rem:
            buf, gs, ss = B[r % 3]
            if r >= 3 * T + 3:
                pltpu.make_async_copy(buf, acc_sh.at[dst_v.at[r - 3]], ss).wait()
                pltpu.async_copy(y_hbm.at[src_v.at[r]], buf, gs)
            pltpu.make_async_copy(y_hbm.at[src_v.at[r]], buf, gs).wait()
            pltpu.async_copy(buf, acc_sh.at[dst_v.at[r]], ss, add=True)
        last = {}
        for r in rem:
            last[r % 3] = r
        for k in sorted(last):
            buf, gs, ss = B[k]
            pltpu.make_async_copy(buf, acc_sh.at[dst_v.at[last[k]]], ss).wait()

    pipe(PH)
    load_idx(PH, PH)
    pipe(PH)
    load_idx(2 * PH, NCH - 2 * PH)
    pipe(NCH - 2 * PH)
    plsc.subcore_barrier()
    for k in range(NPC):
        q = s * NPC + k
        buf, gs = (rows_v, gsa) if k % 2 == 0 else (rows2_v, gsb)
        if k >= 2:
            p = s * NPC + k - 2
            pltpu.make_async_copy(buf, out_hbm.at[c, pl.ds(p * CH, CH)], gs).wait()
        pltpu.sync_copy(acc_sh.at[pl.ds(q * CH, CH)], buf)
        pltpu.async_copy(buf, out_hbm.at[c, pl.ds(q * CH, CH)], gs)
    for k in (NPC - 2, NPC - 1):
        q = s * NPC + k
        buf, gs = (rows_v, gsa) if k % 2 == 0 else (rows2_v, gsb)
        pltpu.make_async_copy(buf, out_hbm.at[c, pl.ds(q * CH, CH)], gs).wait()


def _dinv_from(degp_ref):
    deg = degp_ref[0] + degp_ref[1] + 1.0
    return jnp.where(deg > 0, lax.rsqrt(deg), 0.0)


def _tc1_body(degp_ref, x_ref, w_ref, o_ref):
    dinv = _dinv_from(degp_ref)
    o_ref[...] = dinv * jnp.dot(x_ref[...], w_ref[...],
                                preferred_element_type=jnp.float32)


_tc1 = pl.pallas_call(
    _tc1_body,
    grid=(N // BR,),
    in_specs=[
        pl.BlockSpec((2, BR, 1), lambda i: (0, i, 0)),
        pl.BlockSpec((BR, D), lambda i: (i, 0)),
        pl.BlockSpec((D, D), lambda i: (0, 0)),
    ],
    out_specs=pl.BlockSpec((BR, D), lambda i: (i, 0)),
    out_shape=jax.ShapeDtypeStruct((N, D), jnp.float32),
)


def _tc2_body(degp_ref, s_ref, y1_ref, b1_ref, w2_ref, o_ref):
    dinv = _dinv_from(degp_ref)
    sv = s_ref[...]
    h = jnp.maximum(dinv * (sv[0] + sv[1] + y1_ref[...]) + b1_ref[...], 0.0)
    o_ref[...] = dinv * jnp.dot(h, w2_ref[...],
                                preferred_element_type=jnp.float32)


_tc2 = pl.pallas_call(
    _tc2_body,
    grid=(N // BR,),
    in_specs=[
        pl.BlockSpec((2, BR, 1), lambda i: (0, i, 0)),
        pl.BlockSpec((2, BR, D), lambda i: (0, i, 0)),
        pl.BlockSpec((BR, D), lambda i: (i, 0)),
        pl.BlockSpec((1, D), lambda i: (0, 0)),
        pl.BlockSpec((D, D), lambda i: (0, 0)),
    ],
    out_specs=pl.BlockSpec((BR, D), lambda i: (i, 0)),
    out_shape=jax.ShapeDtypeStruct((N, D), jnp.float32),
)


def _tc3_body(degp_ref, s_ref, y2_ref, b2_ref, o_ref):
    dinv = _dinv_from(degp_ref)
    sv = s_ref[...]
    o_ref[...] = dinv * (sv[0] + sv[1] + y2_ref[...]) + b2_ref[...]


_tc3 = pl.pallas_call(
    _tc3_body,
    grid=(N // BR,),
    in_specs=[
        pl.BlockSpec((2, BR, 1), lambda i: (0, i, 0)),
        pl.BlockSpec((2, BR, D), lambda i: (0, i, 0)),
        pl.BlockSpec((BR, D), lambda i: (i, 0)),
        pl.BlockSpec((1, D), lambda i: (0, 0)),
    ],
    out_specs=pl.BlockSpec((BR, D), lambda i: (i, 0)),
    out_shape=jax.ShapeDtypeStruct((N, D), jnp.float32),
)


def kernel(x, edge_index, W1, b1, W2, b2):
    ei = edge_index.astype(jnp.int32)
    src3 = ei[0].reshape(NW, NCH, CH)
    dst3 = ei[1].reshape(NW, NCH, CH)
    ones128 = jnp.ones((CH, D), jnp.float32)
    z128 = jnp.zeros((CH, D), jnp.float32)

    degp = _deg_kernel(dst3, ones128, z128)
    degc = degp[:, :N, 0:1]
    y1 = _tc1(degc, x, W1)
    s1 = _agg_kernel(y1, src3, dst3, z128)
    y2 = _tc2(degc, s1[:, :N], y1, b1.reshape(1, D), W2)
    s2 = _agg_kernel(y2, src3, dst3, z128)
    out = _tc3(degc, s2[:, :N], y2, b2.reshape(1, D))
    return out
